# baseline (speedup 1.0000x reference)
import functools

import jax
import jax.numpy as jnp
from jax import lax
from jax.experimental import pallas as pl
from jax.experimental.pallas import tpu as pltpu
from jax.experimental.pallas import tpu_sc as plsc

_N = 10000
_E = 320000
_HID = 256
_ALPHA = 0.3
_EPS = 1e-5

_NC = 2
_NS = 16
_CH = 128
_NPAD = 10240
_EP = 327680
_DUMP = 10016
_SEG_CPS = _EP // _CH // _NS
_HIST_CPS = _EP // _CH // _NC // _NS

_B = 1000


def _vsmesh():
    return plsc.VectorSubcoreMesh(core_axis_name="c", subcore_axis_name="s")


def _fill(ref, rows, value):
    @pl.loop(0, rows)
    def _(i):
        ref[pl.ds(i, 1), :] = jnp.full((1, 16), value, jnp.float32)


def _fill_wide(ref, rows, cols, value):
    @pl.loop(0, rows)
    def _(i):
        @pl.loop(0, cols, step=16)
        def _(j):
            ref[pl.ds(i, 1), pl.ds(j, 16)] = jnp.full((1, 16), value, jnp.float32)


def _writeback(acc_sh, out_hbm, c, s):
    @pl.when(s < _NS - 1)
    def _():
        for j in range(5):
            pltpu.sync_copy(
                acc_sh.at[pl.ds(s * 640 + j * 128, 128)],
                out_hbm.at[pl.ds(c * _N + s * 640 + j * 128, 128)],
            )

    @pl.when(s == _NS - 1)
    def _():
        for j in range(3):
            pltpu.sync_copy(
                acc_sh.at[pl.ds(9600 + j * 128, 128)],
                out_hbm.at[pl.ds(c * _N + 9600 + j * 128, 128)],
            )
        pltpu.sync_copy(acc_sh.at[pl.ds(9984, 16)],
                        out_hbm.at[pl.ds(c * _N + 9984, 16)])


def _sc_hist(dst_p):
    @functools.partial(
        pl.kernel,
        out_type=jax.ShapeDtypeStruct((_NC * _N, 16), jnp.float32),
        mesh=_vsmesh(),
        scratch_types=[
            pltpu.VMEM((_CH,), jnp.int32),
            pltpu.VMEM((_CH, 16), jnp.float32),
            pltpu.VMEM((128, 16), jnp.float32),
            pltpu.VMEM_SHARED((_NPAD, 16), jnp.float32),
        ],
    )
    def hist(dst_hbm, out_hbm, idx_v, ones_v, zer_v, acc_sh):
        c = lax.axis_index("c")
        s = lax.axis_index("s")
        _fill(ones_v, _CH, 1.0)
        _fill(zer_v, 128, 0.0)

        @pl.loop(0, 5)
        def _(j):
            pltpu.sync_copy(zer_v, acc_sh.at[pl.ds(s * 640 + j * 128, 128)])

        plsc.subcore_barrier()
        base = (c * _NS + s) * _HIST_CPS

        @pl.loop(0, _HIST_CPS)
        def _(k):
            off = (base + k) * _CH
            pltpu.sync_copy(dst_hbm.at[pl.ds(off, _CH)], idx_v)
            pltpu.sync_copy(ones_v, acc_sh.at[idx_v], add=True)

        plsc.subcore_barrier()
        _writeback(acc_sh, out_hbm, c, s)

    return hist(dst_p)


def _sc_segsum(table2, src0_2d, src1_2d, dst_2d):
    n = _SEG_CPS

    @functools.partial(
        pl.kernel,
        out_type=jax.ShapeDtypeStruct((_NC * _N, 128), jnp.float32),
        mesh=_vsmesh(),
        scratch_types=[
            pltpu.VMEM((_CH,), jnp.int32),
            pltpu.VMEM((_CH,), jnp.int32),
            pltpu.VMEM((_CH, 128), jnp.float32),
            pltpu.VMEM_SHARED((_NPAD, 128), jnp.float32),
        ],
    )
    def segsum(t_hbm, s0_hbm, s1_hbm, d_hbm, out_hbm, src_v, dst_v,
               rows0, acc_sh):
        c = lax.axis_index("c")
        s = lax.axis_index("s")

        _fill_wide(rows0, _CH, 128, 0.0)

        @pl.loop(0, 5)
        def _(j):
            pltpu.sync_copy(rows0, acc_sh.at[pl.ds(s * 640 + j * 128, 128)])

        plsc.subcore_barrier()

        @pl.loop(0, n)
        def _(j):
            off = (s * n + j) * _CH

            @pl.when(c == 0)
            def _():
                pltpu.sync_copy(s0_hbm.at[pl.ds(off, _CH)], src_v)

            @pl.when(c == 1)
            def _():
                pltpu.sync_copy(s1_hbm.at[pl.ds(off, _CH)], src_v)

            pltpu.sync_copy(d_hbm.at[pl.ds(off, _CH)], dst_v)
            pltpu.sync_copy(t_hbm.at[src_v], rows0)
            pltpu.sync_copy(rows0, acc_sh.at[dst_v], add=True)

        plsc.subcore_barrier()
        _writeback(acc_sh, out_hbm, c, s)

    return segsum(table2, src0_2d, src1_2d, dst_2d)


def _dinv_of(degp_ref):
    deg = 1.0 + degp_ref[0, :, 0:1] + degp_ref[1, :, 0:1]
    return lax.rsqrt(deg)


def _tc_proj(x, projW, projB):
    def body(x_ref, w_ref, b_ref, o_ref):
        o_ref[...] = (
            jnp.dot(x_ref[...], w_ref[...], preferred_element_type=jnp.float32)
            + b_ref[...][None, :]
        )

    return pl.pallas_call(
        body,
        grid=(_N // _B,),
        in_specs=[
            pl.BlockSpec((_B, 128), lambda i: (i, 0)),
            pl.BlockSpec((128, _HID), lambda i: (0, 0)),
            pl.BlockSpec((_HID,), lambda i: (0,)),
        ],
        out_specs=pl.BlockSpec((_B, _HID), lambda i: (i, 0)),
        out_shape=jax.ShapeDtypeStruct((_N, _HID), jnp.float32),
    )(x, projW, projB)


def _tc_msg0(h0, convW0, degp):
    def body(h_ref, w_ref, degp_ref, o_ref):
        dinv = _dinv_of(degp_ref)
        hw = jnp.dot(h_ref[...], w_ref[...], preferred_element_type=jnp.float32)
        o_ref[0, :, :] = hw[:, :128] * dinv
        o_ref[1, :, :] = hw[:, 128:] * dinv

    return pl.pallas_call(
        body,
        grid=(_N // _B,),
        in_specs=[
            pl.BlockSpec((_B, _HID), lambda i: (i, 0)),
            pl.BlockSpec((_HID, _HID), lambda i: (0, 0)),
            pl.BlockSpec((2, _B, 16), lambda i: (0, i, 0)),
        ],
        out_specs=pl.BlockSpec((2, _B, 128), lambda i: (0, i, 0)),
        out_shape=jax.ShapeDtypeStruct((2, _N, 128), jnp.float32),
    )(h0, convW0, degp)


def _update_core(s_ref, t_ref, h_ref, h0_ref, degp_ref, cb_ref, nzG_ref, nzB_ref,
                 w1_ref, b1_ref, w2_ref, b2_ref, tau_ref, npG_ref, npB_ref):
    dinv = _dinv_of(degp_ref)
    h = h_ref[...]
    h0 = h0_ref[...]
    t = jnp.concatenate([t_ref[0, :, :], t_ref[1, :, :]], axis=1)
    ssum = jnp.concatenate([s_ref[0, :, :], s_ref[1, :, :]], axis=1)
    raw = dinv * (ssum + t) + cb_ref[...][None, :] + _ALPHA * h0

    mu = (jnp.sum(raw, axis=1, keepdims=True) + jnp.sum(h, axis=1, keepdims=True)) / (2 * _HID)
    dr = raw - mu
    dh = h - mu
    var = (jnp.sum(dr * dr, axis=1, keepdims=True)
           + jnp.sum(dh * dh, axis=1, keepdims=True)) / (2 * _HID)
    rstd = lax.rsqrt(var + _EPS)
    nzG = nzG_ref[...]
    nzB = nzB_ref[...]
    zr = dr * rstd * nzG[None, :_HID] + nzB[None, :_HID]
    zh = dh * rstd * nzG[None, _HID:] + nzB[None, _HID:]

    w1 = w1_ref[...]
    g1 = (jnp.dot(zr, w1[:_HID, :], preferred_element_type=jnp.float32)
          + jnp.dot(zh, w1[_HID:, :], preferred_element_type=jnp.float32)
          + b1_ref[...][None, :])
    g1 = jnp.maximum(g1, 0.0)
    gl = jnp.sum(g1 * w2_ref[...][None, :], axis=1, keepdims=True) + b2_ref[0]
    gate = jax.nn.sigmoid(gl / tau_ref[0])

    hn = gate * raw + (1.0 - gate) * (h + _ALPHA * h0)
    mu2 = jnp.mean(hn, axis=1, keepdims=True)
    d2 = hn - mu2
    var2 = jnp.mean(d2 * d2, axis=1, keepdims=True)
    hn = d2 * lax.rsqrt(var2 + _EPS) * npG_ref[...][None, :] + npB_ref[...][None, :]
    return jnp.maximum(hn, 0.0), dinv


def _layer_in_specs():
    return [
        pl.BlockSpec((2, _B, 128), lambda i: (0, i, 0)),
        pl.BlockSpec((2, _B, 128), lambda i: (0, i, 0)),
        pl.BlockSpec((_B, _HID), lambda i: (i, 0)),
        pl.BlockSpec((_B, _HID), lambda i: (i, 0)),
        pl.BlockSpec((2, _B, 16), lambda i: (0, i, 0)),
        pl.BlockSpec((_HID,), lambda i: (0,)),
        pl.BlockSpec((2 * _HID,), lambda i: (0,)),
        pl.BlockSpec((2 * _HID,), lambda i: (0,)),
        pl.BlockSpec((2 * _HID, 128), lambda i: (0, 0)),
        pl.BlockSpec((128,), lambda i: (0,)),
        pl.BlockSpec((128,), lambda i: (0,)),
        pl.BlockSpec((1,), lambda i: (0,)),
        pl.BlockSpec((1,), lambda i: (0,)),
        pl.BlockSpec((_HID,), lambda i: (0,)),
        pl.BlockSpec((_HID,), lambda i: (0,)),
        pl.BlockSpec((_HID, _HID), lambda i: (0, 0)),
    ]


def _tc_update(s2, t2, h, h0, degp, convBl, nzG, nzB, gateW1, gateB1, gateW2v,
               gateB2, tau1, npG, npB, convW_next):
    def body(s_ref, t_ref, h_ref, h0_ref, degp_ref, cb_ref, nzG_ref, nzB_ref,
             w1_ref, b1_ref, w2_ref, b2_ref, tau_ref, npG_ref, npB_ref,
             wn_ref, oh_ref, ot_ref):
        hr, dinv = _update_core(s_ref, t_ref, h_ref, h0_ref, degp_ref, cb_ref,
                                nzG_ref, nzB_ref, w1_ref, b1_ref, w2_ref,
                                b2_ref, tau_ref, npG_ref, npB_ref)
        oh_ref[...] = hr
        hw = jnp.dot(hr, wn_ref[...], preferred_element_type=jnp.float32)
        ot_ref[0, :, :] = hw[:, :128] * dinv
        ot_ref[1, :, :] = hw[:, 128:] * dinv

    return pl.pallas_call(
        body,
        grid=(_N // _B,),
        in_specs=_layer_in_specs(),
        out_specs=[
            pl.BlockSpec((_B, _HID), lambda i: (i, 0)),
            pl.BlockSpec((2, _B, 128), lambda i: (0, i, 0)),
        ],
        out_shape=[
            jax.ShapeDtypeStruct((_N, _HID), jnp.float32),
            jax.ShapeDtypeStruct((2, _N, 128), jnp.float32),
        ],
    )(s2, t2, h, h0, degp, convBl, nzG, nzB, gateW1, gateB1, gateW2v, gateB2,
      tau1, npG, npB, convW_next)


def _tc_update_final(s2, t2, h, h0, degp, convBl, nzG, nzB, gateW1, gateB1,
                     gateW2v, gateB2, tau1, npG, npB, outW, outB):
    def body(s_ref, t_ref, h_ref, h0_ref, degp_ref, cb_ref, nzG_ref, nzB_ref,
             w1_ref, b1_ref, w2_ref, b2_ref, tau_ref, npG_ref, npB_ref,
             wo_ref, bo_ref, o_ref):
        hr, _ = _update_core(s_ref, t_ref, h_ref, h0_ref, degp_ref, cb_ref,
                             nzG_ref, nzB_ref, w1_ref, b1_ref, w2_ref,
                             b2_ref, tau_ref, npG_ref, npB_ref)
        o_ref[...] = (
            jnp.dot(hr, wo_ref[...], preferred_element_type=jnp.float32)
            + bo_ref[...][None, :]
        )

    in_specs = _layer_in_specs()[:-1] + [
        pl.BlockSpec((_HID, 64), lambda i: (0, 0)),
        pl.BlockSpec((64,), lambda i: (0,)),
    ]
    return pl.pallas_call(
        body,
        grid=(_N // _B,),
        in_specs=in_specs,
        out_specs=pl.BlockSpec((_B, 64), lambda i: (i, 0)),
        out_shape=jax.ShapeDtypeStruct((_N, 64), jnp.float32),
    )(s2, t2, h, h0, degp, convBl, nzG, nzB, gateW1, gateB1, gateW2v, gateB2,
      tau1, npG, npB, outW, outB)


def kernel(x, edge_index, projW, projB, convW, convB, gateW1, gateB1, gateW2,
           gateB2, nzG, nzB, npG, npB, outW, outB, tau):
    src = edge_index[0]
    dst = edge_index[1]
    gateW2v = jnp.reshape(gateW2, (128,))
    tau1 = jnp.reshape(tau, (1,))

    pad = _EP - _E
    src_p = jnp.concatenate([src, jnp.zeros((pad,), jnp.int32)])
    dst_p = jnp.concatenate([dst, jnp.full((pad,), _DUMP, jnp.int32)])
    src0_2d = src_p
    src1_2d = src_p + _N
    dst_2d = dst_p

    degp_flat = _sc_hist(dst_p)
    degp = jnp.reshape(degp_flat, (2, _N, 16))
    h0 = _tc_proj(x, projW, projB)
    t = _tc_msg0(h0, convW[0], degp)

    h = h0
    for l in range(4):
        s2_flat = _sc_segsum(jnp.reshape(t, (2 * _N, 128)), src0_2d, src1_2d, dst_2d)
        s2 = jnp.reshape(s2_flat, (2, _N, 128))
        if l < 3:
            h, t = _tc_update(s2, t, h, h0, degp, convB[l], nzG, nzB, gateW1,
                              gateB1, gateW2v, gateB2, tau1, npG, npB,
                              convW[l + 1])
        else:
            out = _tc_update_final(s2, t, h, h0, degp, convB[l], nzG, nzB,
                                   gateW1, gateB1, gateW2v, gateB2, tau1,
                                   npG, npB, outW, outB)
    return out


# speedup vs baseline: 1.1295x; 1.1295x over previous
import functools

import jax
import jax.numpy as jnp
from jax import lax
from jax.experimental import pallas as pl
from jax.experimental.pallas import tpu as pltpu
from jax.experimental.pallas import tpu_sc as plsc

_N = 10000
_E = 320000
_HID = 256
_ALPHA = 0.3
_EPS = 1e-5

_NC = 2
_NS = 16
_CH = 128
_NPAD = 10240
_EP = 327680
_DUMP = 10016
_SEG_CPS = _EP // _CH // _NS
_HIST_CPS = _EP // _CH // _NC // _NS

_B = 1000


def _vsmesh():
    return plsc.VectorSubcoreMesh(core_axis_name="c", subcore_axis_name="s")


def _fill(ref, rows, value):
    @pl.loop(0, rows)
    def _(i):
        ref[pl.ds(i, 1), :] = jnp.full((1, 16), value, jnp.float32)


def _fill_wide(ref, rows, cols, value):
    @pl.loop(0, rows)
    def _(i):
        @pl.loop(0, cols, step=16)
        def _(j):
            ref[pl.ds(i, 1), pl.ds(j, 16)] = jnp.full((1, 16), value, jnp.float32)


def _writeback(acc_sh, out_hbm, c, s):
    @pl.when(s < _NS - 1)
    def _():
        for j in range(5):
            pltpu.sync_copy(
                acc_sh.at[pl.ds(s * 640 + j * 128, 128)],
                out_hbm.at[pl.ds(c * _N + s * 640 + j * 128, 128)],
            )

    @pl.when(s == _NS - 1)
    def _():
        for j in range(3):
            pltpu.sync_copy(
                acc_sh.at[pl.ds(9600 + j * 128, 128)],
                out_hbm.at[pl.ds(c * _N + 9600 + j * 128, 128)],
            )
        pltpu.sync_copy(acc_sh.at[pl.ds(9984, 16)],
                        out_hbm.at[pl.ds(c * _N + 9984, 16)])


def _sc_hist(dst_p):
    @functools.partial(
        pl.kernel,
        out_type=jax.ShapeDtypeStruct((_NC * _N, 16), jnp.float32),
        mesh=_vsmesh(),
        scratch_types=[
            pltpu.VMEM((_CH,), jnp.int32),
            pltpu.VMEM((_CH, 16), jnp.float32),
            pltpu.VMEM((128, 16), jnp.float32),
            pltpu.VMEM_SHARED((_NPAD, 16), jnp.float32),
        ],
    )
    def hist(dst_hbm, out_hbm, idx_v, ones_v, zer_v, acc_sh):
        c = lax.axis_index("c")
        s = lax.axis_index("s")
        _fill(ones_v, _CH, 1.0)
        _fill(zer_v, 128, 0.0)

        @pl.loop(0, 5)
        def _(j):
            pltpu.sync_copy(zer_v, acc_sh.at[pl.ds(s * 640 + j * 128, 128)])

        plsc.subcore_barrier()
        base = (c * _NS + s) * _HIST_CPS

        @pl.loop(0, _HIST_CPS)
        def _(k):
            off = (base + k) * _CH
            pltpu.sync_copy(dst_hbm.at[pl.ds(off, _CH)], idx_v)
            pltpu.sync_copy(ones_v, acc_sh.at[idx_v], add=True)

        plsc.subcore_barrier()
        _writeback(acc_sh, out_hbm, c, s)

    return hist(dst_p)


def _sc_segsum(table2, src0_2d, src1_2d, dst_2d):
    n = _SEG_CPS

    @functools.partial(
        pl.kernel,
        out_type=jax.ShapeDtypeStruct((_NC * _N, 128), jnp.float32),
        mesh=_vsmesh(),
        scratch_types=[
            pltpu.VMEM((_CH,), jnp.int32),
            pltpu.VMEM((_CH,), jnp.int32),
            pltpu.VMEM((_CH, 128), jnp.float32),
            pltpu.VMEM_SHARED((_NPAD, 128), jnp.float32),
        ],
    )
    def segsum(t_hbm, s0_hbm, s1_hbm, d_hbm, out_hbm, src_v, dst_v,
               rows0, acc_sh):
        c = lax.axis_index("c")
        s = lax.axis_index("s")

        _fill_wide(rows0, _CH, 128, 0.0)

        @pl.loop(0, 5)
        def _(j):
            pltpu.sync_copy(rows0, acc_sh.at[pl.ds(s * 640 + j * 128, 128)])

        plsc.subcore_barrier()

        @pl.loop(0, n)
        def _(j):
            off = (s + j * _NS) * _CH

            @pl.when(c == 0)
            def _():
                pltpu.sync_copy(s0_hbm.at[pl.ds(off, _CH)], src_v)

            @pl.when(c == 1)
            def _():
                pltpu.sync_copy(s1_hbm.at[pl.ds(off, _CH)], src_v)

            pltpu.sync_copy(d_hbm.at[pl.ds(off, _CH)], dst_v)
            pltpu.sync_copy(t_hbm.at[src_v], rows0)
            pltpu.sync_copy(rows0, acc_sh.at[dst_v], add=True)

        plsc.subcore_barrier()
        _writeback(acc_sh, out_hbm, c, s)

    return segsum(table2, src0_2d, src1_2d, dst_2d)


def _dinv_of(degp_ref):
    deg = 1.0 + degp_ref[0, :, 0:1] + degp_ref[1, :, 0:1]
    return lax.rsqrt(deg)


def _tc_proj(x, projW, projB):
    def body(x_ref, w_ref, b_ref, o_ref):
        o_ref[...] = (
            jnp.dot(x_ref[...], w_ref[...], preferred_element_type=jnp.float32)
            + b_ref[...][None, :]
        )

    return pl.pallas_call(
        body,
        grid=(_N // _B,),
        in_specs=[
            pl.BlockSpec((_B, 128), lambda i: (i, 0)),
            pl.BlockSpec((128, _HID), lambda i: (0, 0)),
            pl.BlockSpec((_HID,), lambda i: (0,)),
        ],
        out_specs=pl.BlockSpec((_B, _HID), lambda i: (i, 0)),
        out_shape=jax.ShapeDtypeStruct((_N, _HID), jnp.float32),
    )(x, projW, projB)


def _tc_msg0(h0, convW0, degp):
    def body(h_ref, w_ref, degp_ref, o_ref):
        dinv = _dinv_of(degp_ref)
        hw = jnp.dot(h_ref[...], w_ref[...], preferred_element_type=jnp.float32)
        o_ref[0, :, :] = hw[:, :128] * dinv
        o_ref[1, :, :] = hw[:, 128:] * dinv

    return pl.pallas_call(
        body,
        grid=(_N // _B,),
        in_specs=[
            pl.BlockSpec((_B, _HID), lambda i: (i, 0)),
            pl.BlockSpec((_HID, _HID), lambda i: (0, 0)),
            pl.BlockSpec((2, _B, 16), lambda i: (0, i, 0)),
        ],
        out_specs=pl.BlockSpec((2, _B, 128), lambda i: (0, i, 0)),
        out_shape=jax.ShapeDtypeStruct((2, _N, 128), jnp.float32),
    )(h0, convW0, degp)


def _update_core(s_ref, t_ref, h_ref, h0_ref, degp_ref, cb_ref, nzG_ref, nzB_ref,
                 w1_ref, b1_ref, w2_ref, b2_ref, tau_ref, npG_ref, npB_ref):
    dinv = _dinv_of(degp_ref)
    h = h_ref[...]
    h0 = h0_ref[...]
    t = jnp.concatenate([t_ref[0, :, :], t_ref[1, :, :]], axis=1)
    ssum = jnp.concatenate([s_ref[0, :, :], s_ref[1, :, :]], axis=1)
    raw = dinv * (ssum + t) + cb_ref[...][None, :] + _ALPHA * h0

    mu = (jnp.sum(raw, axis=1, keepdims=True) + jnp.sum(h, axis=1, keepdims=True)) / (2 * _HID)
    dr = raw - mu
    dh = h - mu
    var = (jnp.sum(dr * dr, axis=1, keepdims=True)
           + jnp.sum(dh * dh, axis=1, keepdims=True)) / (2 * _HID)
    rstd = lax.rsqrt(var + _EPS)
    nzG = nzG_ref[...]
    nzB = nzB_ref[...]
    zr = dr * rstd * nzG[None, :_HID] + nzB[None, :_HID]
    zh = dh * rstd * nzG[None, _HID:] + nzB[None, _HID:]

    w1 = w1_ref[...]
    g1 = (jnp.dot(zr, w1[:_HID, :], preferred_element_type=jnp.float32)
          + jnp.dot(zh, w1[_HID:, :], preferred_element_type=jnp.float32)
          + b1_ref[...][None, :])
    g1 = jnp.maximum(g1, 0.0)
    gl = jnp.sum(g1 * w2_ref[...][None, :], axis=1, keepdims=True) + b2_ref[0]
    gate = jax.nn.sigmoid(gl / tau_ref[0])

    hn = gate * raw + (1.0 - gate) * (h + _ALPHA * h0)
    mu2 = jnp.mean(hn, axis=1, keepdims=True)
    d2 = hn - mu2
    var2 = jnp.mean(d2 * d2, axis=1, keepdims=True)
    hn = d2 * lax.rsqrt(var2 + _EPS) * npG_ref[...][None, :] + npB_ref[...][None, :]
    return jnp.maximum(hn, 0.0), dinv


def _layer_in_specs():
    return [
        pl.BlockSpec((2, _B, 128), lambda i: (0, i, 0)),
        pl.BlockSpec((2, _B, 128), lambda i: (0, i, 0)),
        pl.BlockSpec((_B, _HID), lambda i: (i, 0)),
        pl.BlockSpec((_B, _HID), lambda i: (i, 0)),
        pl.BlockSpec((2, _B, 16), lambda i: (0, i, 0)),
        pl.BlockSpec((_HID,), lambda i: (0,)),
        pl.BlockSpec((2 * _HID,), lambda i: (0,)),
        pl.BlockSpec((2 * _HID,), lambda i: (0,)),
        pl.BlockSpec((2 * _HID, 128), lambda i: (0, 0)),
        pl.BlockSpec((128,), lambda i: (0,)),
        pl.BlockSpec((128,), lambda i: (0,)),
        pl.BlockSpec((1,), lambda i: (0,)),
        pl.BlockSpec((1,), lambda i: (0,)),
        pl.BlockSpec((_HID,), lambda i: (0,)),
        pl.BlockSpec((_HID,), lambda i: (0,)),
        pl.BlockSpec((_HID, _HID), lambda i: (0, 0)),
    ]


def _tc_update(s2, t2, h, h0, degp, convBl, nzG, nzB, gateW1, gateB1, gateW2v,
               gateB2, tau1, npG, npB, convW_next):
    def body(s_ref, t_ref, h_ref, h0_ref, degp_ref, cb_ref, nzG_ref, nzB_ref,
             w1_ref, b1_ref, w2_ref, b2_ref, tau_ref, npG_ref, npB_ref,
             wn_ref, oh_ref, ot_ref):
        hr, dinv = _update_core(s_ref, t_ref, h_ref, h0_ref, degp_ref, cb_ref,
                                nzG_ref, nzB_ref, w1_ref, b1_ref, w2_ref,
                                b2_ref, tau_ref, npG_ref, npB_ref)
        oh_ref[...] = hr
        hw = jnp.dot(hr, wn_ref[...], preferred_element_type=jnp.float32)
        ot_ref[0, :, :] = hw[:, :128] * dinv
        ot_ref[1, :, :] = hw[:, 128:] * dinv

    return pl.pallas_call(
        body,
        grid=(_N // _B,),
        in_specs=_layer_in_specs(),
        out_specs=[
            pl.BlockSpec((_B, _HID), lambda i: (i, 0)),
            pl.BlockSpec((2, _B, 128), lambda i: (0, i, 0)),
        ],
        out_shape=[
            jax.ShapeDtypeStruct((_N, _HID), jnp.float32),
            jax.ShapeDtypeStruct((2, _N, 128), jnp.float32),
        ],
    )(s2, t2, h, h0, degp, convBl, nzG, nzB, gateW1, gateB1, gateW2v, gateB2,
      tau1, npG, npB, convW_next)


def _tc_update_final(s2, t2, h, h0, degp, convBl, nzG, nzB, gateW1, gateB1,
                     gateW2v, gateB2, tau1, npG, npB, outW, outB):
    def body(s_ref, t_ref, h_ref, h0_ref, degp_ref, cb_ref, nzG_ref, nzB_ref,
             w1_ref, b1_ref, w2_ref, b2_ref, tau_ref, npG_ref, npB_ref,
             wo_ref, bo_ref, o_ref):
        hr, _ = _update_core(s_ref, t_ref, h_ref, h0_ref, degp_ref, cb_ref,
                             nzG_ref, nzB_ref, w1_ref, b1_ref, w2_ref,
                             b2_ref, tau_ref, npG_ref, npB_ref)
        o_ref[...] = (
            jnp.dot(hr, wo_ref[...], preferred_element_type=jnp.float32)
            + bo_ref[...][None, :]
        )

    in_specs = _layer_in_specs()[:-1] + [
        pl.BlockSpec((_HID, 64), lambda i: (0, 0)),
        pl.BlockSpec((64,), lambda i: (0,)),
    ]
    return pl.pallas_call(
        body,
        grid=(_N // _B,),
        in_specs=in_specs,
        out_specs=pl.BlockSpec((_B, 64), lambda i: (i, 0)),
        out_shape=jax.ShapeDtypeStruct((_N, 64), jnp.float32),
    )(s2, t2, h, h0, degp, convBl, nzG, nzB, gateW1, gateB1, gateW2v, gateB2,
      tau1, npG, npB, outW, outB)


def kernel(x, edge_index, projW, projB, convW, convB, gateW1, gateB1, gateW2,
           gateB2, nzG, nzB, npG, npB, outW, outB, tau):
    src = edge_index[0]
    dst = edge_index[1]
    gateW2v = jnp.reshape(gateW2, (128,))
    tau1 = jnp.reshape(tau, (1,))

    pad = _EP - _E
    src_p = jnp.concatenate([src, jnp.zeros((pad,), jnp.int32)])
    dst_p = jnp.concatenate([dst, jnp.full((pad,), _DUMP, jnp.int32)])
    src0_2d = src_p
    src1_2d = src_p + _N
    dst_2d = dst_p

    degp_flat = _sc_hist(dst_p)
    degp = jnp.reshape(degp_flat, (2, _N, 16))
    h0 = _tc_proj(x, projW, projB)
    t = _tc_msg0(h0, convW[0], degp)

    h = h0
    for l in range(4):
        s2_flat = _sc_segsum(jnp.reshape(t, (2 * _N, 128)), src0_2d, src1_2d, dst_2d)
        s2 = jnp.reshape(s2_flat, (2, _N, 128))
        if l < 3:
            h, t = _tc_update(s2, t, h, h0, degp, convB[l], nzG, nzB, gateW1,
                              gateB1, gateW2v, gateB2, tau1, npG, npB,
                              convW[l + 1])
        else:
            out = _tc_update_final(s2, t, h, h0, degp, convB[l], nzG, nzB,
                                   gateW1, gateB1, gateW2v, gateB2, tau1,
                                   npG, npB, outW, outB)
    return out


# speedup vs baseline: 1.7114x; 1.5152x over previous
import functools

import jax
import jax.numpy as jnp
from jax import lax
from jax.experimental import pallas as pl
from jax.experimental.pallas import tpu as pltpu
from jax.experimental.pallas import tpu_sc as plsc

_N = 10000
_E = 320000
_HID = 256
_ALPHA = 0.3
_EPS = 1e-5

_NC = 2
_NS = 16
_CH = 128
_NPAD = 10240
_EP = 327680
_DUMP = 10016
_SEG_CPS = _EP // _CH // _NS
_HIST_CPS = _EP // _CH // _NC // _NS

_B = 1000


def _vsmesh():
    return plsc.VectorSubcoreMesh(core_axis_name="c", subcore_axis_name="s")


def _fill(ref, rows, value):
    @pl.loop(0, rows)
    def _(i):
        ref[pl.ds(i, 1), :] = jnp.full((1, 16), value, jnp.float32)


def _fill_wide(ref, rows, cols, value):
    @pl.loop(0, rows)
    def _(i):
        @pl.loop(0, cols, step=16)
        def _(j):
            ref[pl.ds(i, 1), pl.ds(j, 16)] = jnp.full((1, 16), value, jnp.float32)


def _writeback(acc_sh, out_hbm, c, s):
    @pl.when(s < _NS - 1)
    def _():
        for j in range(5):
            pltpu.sync_copy(
                acc_sh.at[pl.ds(s * 640 + j * 128, 128)],
                out_hbm.at[pl.ds(c * _N + s * 640 + j * 128, 128)],
            )

    @pl.when(s == _NS - 1)
    def _():
        for j in range(3):
            pltpu.sync_copy(
                acc_sh.at[pl.ds(9600 + j * 128, 128)],
                out_hbm.at[pl.ds(c * _N + 9600 + j * 128, 128)],
            )
        pltpu.sync_copy(acc_sh.at[pl.ds(9984, 16)],
                        out_hbm.at[pl.ds(c * _N + 9984, 16)])


def _sc_hist(dst_p):
    @functools.partial(
        pl.kernel,
        out_type=jax.ShapeDtypeStruct((_NC * _N, 16), jnp.float32),
        mesh=_vsmesh(),
        scratch_types=[
            pltpu.VMEM((_CH,), jnp.int32),
            pltpu.VMEM((_CH, 16), jnp.float32),
            pltpu.VMEM((128, 16), jnp.float32),
            pltpu.VMEM_SHARED((_NPAD, 16), jnp.float32),
        ],
    )
    def hist(dst_hbm, out_hbm, idx_v, ones_v, zer_v, acc_sh):
        c = lax.axis_index("c")
        s = lax.axis_index("s")
        _fill(ones_v, _CH, 1.0)
        _fill(zer_v, 128, 0.0)

        @pl.loop(0, 5)
        def _(j):
            pltpu.sync_copy(zer_v, acc_sh.at[pl.ds(s * 640 + j * 128, 128)])

        plsc.subcore_barrier()
        base = (c * _NS + s) * _HIST_CPS

        @pl.loop(0, _HIST_CPS)
        def _(k):
            off = (base + k) * _CH
            pltpu.sync_copy(dst_hbm.at[pl.ds(off, _CH)], idx_v)
            pltpu.sync_copy(ones_v, acc_sh.at[idx_v], add=True)

        plsc.subcore_barrier()
        _writeback(acc_sh, out_hbm, c, s)

    return hist(dst_p)


def _sc_segsum(table2, src0_2d, dst_2d):
    n_chunks = _E // _CH
    max_iters = (n_chunks + _NS - 1) // _NS

    @functools.partial(
        pl.kernel,
        out_type=jax.ShapeDtypeStruct((_NC * _N, 128), jnp.float32),
        mesh=_vsmesh(),
        scratch_types=[
            pltpu.VMEM((_CH,), jnp.int32),
            pltpu.VMEM((_CH,), jnp.int32),
            pltpu.VMEM((_CH, 128), jnp.float32),
            pltpu.VMEM((128, 128), jnp.float32),
            pltpu.VMEM_SHARED((_NPAD, 128), jnp.float32),
        ],
    )
    def segsum(t_hbm, src_hbm, d_hbm, out_hbm, src_v, dst_v, rows_v, zer_v,
               acc_sh):
        c = lax.axis_index("c")
        s = lax.axis_index("s")
        _fill_wide(zer_v, 128, 128, 0.0)

        @pl.loop(0, 5)
        def _(j):
            pltpu.sync_copy(zer_v, acc_sh.at[pl.ds(s * 640 + j * 128, 128)])

        plsc.subcore_barrier()

        @pl.loop(0, max_iters)
        def _(k):
            idx = s + k * _NS

            @pl.when(idx < n_chunks)
            def _():
                off = idx * _CH
                pltpu.sync_copy(src_hbm.at[pl.ds(off, _CH)], src_v)
                pltpu.sync_copy(d_hbm.at[pl.ds(off, _CH)], dst_v)

                @pl.loop(0, _CH, step=16)
                def _(j):
                    src_v[pl.ds(j, 16)] = src_v[pl.ds(j, 16)] + c * _N

                pltpu.sync_copy(t_hbm.at[src_v], rows_v)
                pltpu.sync_copy(rows_v, acc_sh.at[dst_v], add=True)

        plsc.subcore_barrier()
        _writeback(acc_sh, out_hbm, c, s)

    return segsum(table2, src0_2d, dst_2d)


def _dinv_of(degp_ref):
    deg = 1.0 + degp_ref[0, :, 0:1] + degp_ref[1, :, 0:1]
    return lax.rsqrt(deg)


def _tc_proj(x, projW, projB):
    def body(x_ref, w_ref, b_ref, o_ref):
        o_ref[...] = (
            jnp.dot(x_ref[...], w_ref[...], preferred_element_type=jnp.float32)
            + b_ref[...][None, :]
        )

    return pl.pallas_call(
        body,
        grid=(_N // _B,),
        in_specs=[
            pl.BlockSpec((_B, 128), lambda i: (i, 0)),
            pl.BlockSpec((128, _HID), lambda i: (0, 0)),
            pl.BlockSpec((_HID,), lambda i: (0,)),
        ],
        out_specs=pl.BlockSpec((_B, _HID), lambda i: (i, 0)),
        out_shape=jax.ShapeDtypeStruct((_N, _HID), jnp.float32),
    )(x, projW, projB)


def _tc_msg0(h0, convW0, degp):
    def body(h_ref, w_ref, degp_ref, o_ref):
        dinv = _dinv_of(degp_ref)
        hw = jnp.dot(h_ref[...], w_ref[...], preferred_element_type=jnp.float32)
        o_ref[0, :, :] = hw[:, :128] * dinv
        o_ref[1, :, :] = hw[:, 128:] * dinv

    return pl.pallas_call(
        body,
        grid=(_N // _B,),
        in_specs=[
            pl.BlockSpec((_B, _HID), lambda i: (i, 0)),
            pl.BlockSpec((_HID, _HID), lambda i: (0, 0)),
            pl.BlockSpec((2, _B, 16), lambda i: (0, i, 0)),
        ],
        out_specs=pl.BlockSpec((2, _B, 128), lambda i: (0, i, 0)),
        out_shape=jax.ShapeDtypeStruct((2, _N, 128), jnp.float32),
    )(h0, convW0, degp)


def _update_core(s_ref, t_ref, h_ref, h0_ref, degp_ref, cb_ref, nzG_ref, nzB_ref,
                 w1_ref, b1_ref, w2_ref, b2_ref, tau_ref, npG_ref, npB_ref):
    dinv = _dinv_of(degp_ref)
    h = h_ref[...]
    h0 = h0_ref[...]
    t = jnp.concatenate([t_ref[0, :, :], t_ref[1, :, :]], axis=1)
    ssum = jnp.concatenate([s_ref[0, :, :], s_ref[1, :, :]], axis=1)
    raw = dinv * (ssum + t) + cb_ref[...][None, :] + _ALPHA * h0

    mu = (jnp.sum(raw, axis=1, keepdims=True) + jnp.sum(h, axis=1, keepdims=True)) / (2 * _HID)
    dr = raw - mu
    dh = h - mu
    var = (jnp.sum(dr * dr, axis=1, keepdims=True)
           + jnp.sum(dh * dh, axis=1, keepdims=True)) / (2 * _HID)
    rstd = lax.rsqrt(var + _EPS)
    nzG = nzG_ref[...]
    nzB = nzB_ref[...]
    zr = dr * rstd * nzG[None, :_HID] + nzB[None, :_HID]
    zh = dh * rstd * nzG[None, _HID:] + nzB[None, _HID:]

    w1 = w1_ref[...]
    g1 = (jnp.dot(zr, w1[:_HID, :], preferred_element_type=jnp.float32)
          + jnp.dot(zh, w1[_HID:, :], preferred_element_type=jnp.float32)
          + b1_ref[...][None, :])
    g1 = jnp.maximum(g1, 0.0)
    gl = jnp.sum(g1 * w2_ref[...][None, :], axis=1, keepdims=True) + b2_ref[0]
    gate = jax.nn.sigmoid(gl / tau_ref[0])

    hn = gate * raw + (1.0 - gate) * (h + _ALPHA * h0)
    mu2 = jnp.mean(hn, axis=1, keepdims=True)
    d2 = hn - mu2
    var2 = jnp.mean(d2 * d2, axis=1, keepdims=True)
    hn = d2 * lax.rsqrt(var2 + _EPS) * npG_ref[...][None, :] + npB_ref[...][None, :]
    return jnp.maximum(hn, 0.0), dinv


def _layer_in_specs():
    return [
        pl.BlockSpec((2, _B, 128), lambda i: (0, i, 0)),
        pl.BlockSpec((2, _B, 128), lambda i: (0, i, 0)),
        pl.BlockSpec((_B, _HID), lambda i: (i, 0)),
        pl.BlockSpec((_B, _HID), lambda i: (i, 0)),
        pl.BlockSpec((2, _B, 16), lambda i: (0, i, 0)),
        pl.BlockSpec((_HID,), lambda i: (0,)),
        pl.BlockSpec((2 * _HID,), lambda i: (0,)),
        pl.BlockSpec((2 * _HID,), lambda i: (0,)),
        pl.BlockSpec((2 * _HID, 128), lambda i: (0, 0)),
        pl.BlockSpec((128,), lambda i: (0,)),
        pl.BlockSpec((128,), lambda i: (0,)),
        pl.BlockSpec((1,), lambda i: (0,)),
        pl.BlockSpec((1,), lambda i: (0,)),
        pl.BlockSpec((_HID,), lambda i: (0,)),
        pl.BlockSpec((_HID,), lambda i: (0,)),
        pl.BlockSpec((_HID, _HID), lambda i: (0, 0)),
    ]


def _tc_update(s2, t2, h, h0, degp, convBl, nzG, nzB, gateW1, gateB1, gateW2v,
               gateB2, tau1, npG, npB, convW_next):
    def body(s_ref, t_ref, h_ref, h0_ref, degp_ref, cb_ref, nzG_ref, nzB_ref,
             w1_ref, b1_ref, w2_ref, b2_ref, tau_ref, npG_ref, npB_ref,
             wn_ref, oh_ref, ot_ref):
        hr, dinv = _update_core(s_ref, t_ref, h_ref, h0_ref, degp_ref, cb_ref,
                                nzG_ref, nzB_ref, w1_ref, b1_ref, w2_ref,
                                b2_ref, tau_ref, npG_ref, npB_ref)
        oh_ref[...] = hr
        hw = jnp.dot(hr, wn_ref[...], preferred_element_type=jnp.float32)
        ot_ref[0, :, :] = hw[:, :128] * dinv
        ot_ref[1, :, :] = hw[:, 128:] * dinv

    return pl.pallas_call(
        body,
        grid=(_N // _B,),
        in_specs=_layer_in_specs(),
        out_specs=[
            pl.BlockSpec((_B, _HID), lambda i: (i, 0)),
            pl.BlockSpec((2, _B, 128), lambda i: (0, i, 0)),
        ],
        out_shape=[
            jax.ShapeDtypeStruct((_N, _HID), jnp.float32),
            jax.ShapeDtypeStruct((2, _N, 128), jnp.float32),
        ],
    )(s2, t2, h, h0, degp, convBl, nzG, nzB, gateW1, gateB1, gateW2v, gateB2,
      tau1, npG, npB, convW_next)


def _tc_update_final(s2, t2, h, h0, degp, convBl, nzG, nzB, gateW1, gateB1,
                     gateW2v, gateB2, tau1, npG, npB, outW, outB):
    def body(s_ref, t_ref, h_ref, h0_ref, degp_ref, cb_ref, nzG_ref, nzB_ref,
             w1_ref, b1_ref, w2_ref, b2_ref, tau_ref, npG_ref, npB_ref,
             wo_ref, bo_ref, o_ref):
        hr, _ = _update_core(s_ref, t_ref, h_ref, h0_ref, degp_ref, cb_ref,
                             nzG_ref, nzB_ref, w1_ref, b1_ref, w2_ref,
                             b2_ref, tau_ref, npG_ref, npB_ref)
        o_ref[...] = (
            jnp.dot(hr, wo_ref[...], preferred_element_type=jnp.float32)
            + bo_ref[...][None, :]
        )

    in_specs = _layer_in_specs()[:-1] + [
        pl.BlockSpec((_HID, 64), lambda i: (0, 0)),
        pl.BlockSpec((64,), lambda i: (0,)),
    ]
    return pl.pallas_call(
        body,
        grid=(_N // _B,),
        in_specs=in_specs,
        out_specs=pl.BlockSpec((_B, 64), lambda i: (i, 0)),
        out_shape=jax.ShapeDtypeStruct((_N, 64), jnp.float32),
    )(s2, t2, h, h0, degp, convBl, nzG, nzB, gateW1, gateB1, gateW2v, gateB2,
      tau1, npG, npB, outW, outB)


def kernel(x, edge_index, projW, projB, convW, convB, gateW1, gateB1, gateW2,
           gateB2, nzG, nzB, npG, npB, outW, outB, tau):
    src = edge_index[0]
    dst = edge_index[1]
    gateW2v = jnp.reshape(gateW2, (128,))
    tau1 = jnp.reshape(tau, (1,))

    pad = _EP - _E
    src_p = jnp.concatenate([src, jnp.zeros((pad,), jnp.int32)])
    dst_p = jnp.concatenate([dst, jnp.full((pad,), _DUMP, jnp.int32)])
    src0_2d = src
    dst_2d = dst

    degp_flat = _sc_hist(dst_p)
    degp = jnp.reshape(degp_flat, (2, _N, 16))
    h0 = _tc_proj(x, projW, projB)
    t = _tc_msg0(h0, convW[0], degp)

    h = h0
    for l in range(4):
        s2_flat = _sc_segsum(jnp.reshape(t, (2 * _N, 128)), src0_2d, dst_2d)
        s2 = jnp.reshape(s2_flat, (2, _N, 128))
        if l < 3:
            h, t = _tc_update(s2, t, h, h0, degp, convB[l], nzG, nzB, gateW1,
                              gateB1, gateW2v, gateB2, tau1, npG, npB,
                              convW[l + 1])
        else:
            out = _tc_update_final(s2, t, h, h0, degp, convB[l], nzG, nzB,
                                   gateW1, gateB1, gateW2v, gateB2, tau1,
                                   npG, npB, outW, outB)
    return out
